# baseline (speedup 1.0000x reference)
import functools

import jax
import jax.numpy as jnp
from jax import lax
from jax.experimental import pallas as pl
from jax.experimental.pallas import tpu as pltpu
from jax.experimental.pallas import tpu_sc as plsc

_B, _T = 4096, 200
_V, _D = 1000000, 32
_NW = 32
_BB = 128
_NB = 4


def _build():
  mesh = plsc.VectorSubcoreMesh(core_axis_name="c", subcore_axis_name="s")
  per_w = _BB * _T

  @functools.partial(
      pl.kernel,
      mesh=mesh,
      out_type=jax.ShapeDtypeStruct((_T, 4, _NW, 8, 128), jnp.float32),
      scratch_types=[
          pltpu.VMEM((per_w,), jnp.int32),
          pltpu.VMEM((_NB, _BB, 128), jnp.float32),
          pltpu.VMEM((_NB, 4, 8, 128), jnp.float32),
          pltpu.VMEM((_NB, _BB), jnp.int32),
          pltpu.VMEM((_NB, _BB), jnp.int32),
          pltpu.SemaphoreType.DMA((_NB,)),
          pltpu.SemaphoreType.DMA((_NB,)),
      ],
      compiler_params=pltpu.CompilerParams(
          use_tc_tiling_on_sc=True, needs_layout_passes=False),
  )
  def emb(idx_hbm, w_hbm, out_hbm, idx_v, g_v, tr_v, il_v, sub_v, gsem, ssem):
    wid = lax.axis_index("s") * 2 + lax.axis_index("c")
    pltpu.sync_copy(idx_hbm.at[pl.ds(wid * per_w, per_w)], idx_v)
    lanes = lax.iota(jnp.int32, 16)

    def gcp(b):
      return pltpu.make_async_copy(
          w_hbm.at[il_v.at[b]], g_v.at[b], gsem.at[b])

    def scp(t, b):
      return pltpu.make_async_copy(
          tr_v.at[b], out_hbm.at[t, :, wid], ssem.at[b])

    def extract(t, b):
      for k in range(8):
        pos = (k * 16 + lanes) * _T + t
        idx = plsc.load_gather(idx_v, [pos])
        il_v[b, pl.ds(k * 16, 16)] = lax.shift_right_logical(idx, 2)
        sub_v[b, pl.ds(k * 16, 16)] = lax.shift_left(
            jnp.bitwise_and(idx, 3), 5)

    def transpose(b):
      def body(d, carry):
        dblk = d // 8
        dr = d % 8
        for k in range(8):
          rows = k * 16 + lanes
          cols = plsc.load_gather(sub_v.at[b], [rows]) + d
          tr_v[b, dblk, dr, pl.ds(k * 16, 16)] = plsc.load_gather(
              g_v.at[b], [rows, cols])
        return carry
      lax.fori_loop(0, _D, body, 0)

    for b in range(_NB):
      extract(b, b)
      gcp(b).start()

    def main_body(r, carry):
      for b in range(_NB):
        t_new = r * _NB + b
        t_old = t_new - _NB
        gcp(b).wait()

        @pl.when(r >= 2)
        def _():
          scp(t_old, b).wait()

        transpose(b)
        scp(t_old, b).start()
        extract(t_new, b)
        gcp(b).start()
      return carry

    lax.fori_loop(1, _T // _NB, main_body, 0)

    for b in range(_NB):
      t_old = _T - _NB + b
      gcp(b).wait()
      scp(t_old, b).wait()
      transpose(b)
      scp(t_old, b).start()
    for b in range(_NB):
      scp(_T - _NB + b, b).wait()

  return emb


@jax.jit
def kernel(x, weight):
  flat = x.reshape(-1)
  out5 = _build()(flat, weight.reshape(_V // 4, 128))
  return out5.transpose(2, 4, 0, 1, 3).reshape(_B, _T, _D)


# speedup vs baseline: 1.2702x; 1.2702x over previous
import functools

import jax
import jax.numpy as jnp
from jax import lax
from jax.experimental import pallas as pl
from jax.experimental.pallas import tpu as pltpu
from jax.experimental.pallas import tpu_sc as plsc

_B, _T = 4096, 200
_V, _D = 1000000, 32
_NW = 32
_BB = 128
_NB = 4


def _build():
  mesh = plsc.VectorSubcoreMesh(core_axis_name="c", subcore_axis_name="s")
  per_w = _BB * _T

  @functools.partial(
      pl.kernel,
      mesh=mesh,
      out_type=jax.ShapeDtypeStruct((_T, 4, _NW, 8, 128), jnp.float32),
      scratch_types=[
          pltpu.VMEM((per_w,), jnp.int32),
          pltpu.VMEM((_NB, _BB, _D), jnp.float32),
          pltpu.VMEM((_NB, 4, 8, 128), jnp.float32),
          pltpu.VMEM((_NB, _BB), jnp.int32),
          pltpu.SemaphoreType.DMA((_NB,)),
          pltpu.SemaphoreType.DMA((_NB,)),
      ],
      compiler_params=pltpu.CompilerParams(
          use_tc_tiling_on_sc=False, needs_layout_passes=False),
  )
  def emb(idx_hbm, w_hbm, out_hbm, idx_v, g_v, tr_v, il_v, gsem, ssem):
    wid = lax.axis_index("s") * 2 + lax.axis_index("c")
    pltpu.sync_copy(idx_hbm.at[pl.ds(wid * per_w, per_w)], idx_v)
    lanes = lax.iota(jnp.int32, 16)

    def gcp(b):
      return pltpu.make_async_copy(
          w_hbm.at[il_v.at[b]], g_v.at[b], gsem.at[b])

    def scp(t, b):
      return pltpu.make_async_copy(
          tr_v.at[b], out_hbm.at[t, :, wid], ssem.at[b])

    def extract(t, b):
      for k in range(8):
        pos = (k * 16 + lanes) * _T + t
        il_v[b, pl.ds(k * 16, 16)] = plsc.load_gather(idx_v, [pos])

    def transpose(b):
      for d in range(_D):
        cols = jnp.zeros((16,), jnp.int32) + d
        for k in range(8):
          rows = k * 16 + lanes
          tr_v[b, d // 8, d % 8, pl.ds(k * 16, 16)] = plsc.load_gather(
              g_v.at[b], [rows, cols])

    for b in range(_NB):
      extract(b, b)
      gcp(b).start()

    def main_body(r, carry):
      for b in range(_NB):
        t_new = r * _NB + b
        t_old = t_new - _NB
        gcp(b).wait()

        @pl.when(r >= 2)
        def _():
          scp(t_old, b).wait()

        transpose(b)
        scp(t_old, b).start()
        extract(t_new, b)
        gcp(b).start()
      return carry

    lax.fori_loop(1, _T // _NB, main_body, 0)

    for b in range(_NB):
      t_old = _T - _NB + b
      gcp(b).wait()
      scp(t_old, b).wait()
      transpose(b)
      scp(t_old, b).start()
    for b in range(_NB):
      scp(_T - _NB + b, b).wait()

  return emb


@jax.jit
def kernel(x, weight):
  flat = x.reshape(-1)
  out5 = _build()(flat, weight)
  return out5.transpose(2, 4, 0, 1, 3).reshape(_B, _T, _D)


# speedup vs baseline: 2.0510x; 1.6146x over previous
import functools

import jax
import jax.numpy as jnp
from jax import lax
from jax.experimental import pallas as pl
from jax.experimental.pallas import tpu as pltpu
from jax.experimental.pallas import tpu_sc as plsc

_B, _T = 4096, 200
_V, _D = 1000000, 32
_NW = 32
_BB = 128
_NB = 4


def _build():
  mesh = plsc.VectorSubcoreMesh(core_axis_name="c", subcore_axis_name="s")
  per_w = _BB * _T

  @functools.partial(
      pl.kernel,
      mesh=mesh,
      out_type=jax.ShapeDtypeStruct((_T, 4, _NW, 8, 128), jnp.float32),
      scratch_types=[
          pltpu.VMEM((per_w,), jnp.int32),
          pltpu.VMEM((_NB, _BB, _D), jnp.float32),
          pltpu.VMEM((_NB, 4, 8, 128), jnp.float32),
          pltpu.VMEM((_NB, _BB), jnp.int32),
          pltpu.SemaphoreType.DMA((_NB,)),
          pltpu.SemaphoreType.DMA((_NB,)),
      ],
      compiler_params=pltpu.CompilerParams(
          use_tc_tiling_on_sc=False, needs_layout_passes=False),
  )
  def emb(idx_hbm, w_hbm, out_hbm, idx_v, g_v, tr_v, il_v, gsem, ssem):
    wid = lax.axis_index("s") * 2 + lax.axis_index("c")
    pltpu.sync_copy(idx_hbm.at[pl.ds(wid * per_w, per_w)], idx_v)
    lanes = lax.iota(jnp.int32, 16)

    def gcp(b):
      return pltpu.make_async_copy(
          w_hbm.at[il_v.at[b]], g_v.at[b], gsem.at[b])

    def scp(t, b):
      return pltpu.make_async_copy(
          tr_v.at[b], out_hbm.at[t, :, wid], ssem.at[b])

    def extract(t, b):
      for k in range(8):
        pos = (k * 16 + lanes) * _T + t
        il_v[b, pl.ds(k * 16, 16)] = plsc.load_gather(idx_v, [pos])

    def transpose(b):
      def body(d0, carry):
        dvec = jnp.bitwise_and(d0 + lanes, _D - 1)
        i0 = lax.shift_right_logical(dvec, 3)
        i1 = jnp.bitwise_and(dvec, 7)
        for k in range(8):
          jvec = k * 16 + lanes
          vals = plsc.load_gather(g_v.at[b], [jvec, dvec])
          plsc.store_scatter(tr_v.at[b], [i0, i1, jvec], vals)
        return carry
      lax.fori_loop(0, _D, body, 0)

    for b in range(_NB):
      extract(b, b)
      gcp(b).start()

    def main_body(r, carry):
      for b in range(_NB):
        t_new = r * _NB + b
        t_old = t_new - _NB
        gcp(b).wait()

        @pl.when(r >= 2)
        def _():
          scp(t_old, b).wait()

        transpose(b)
        scp(t_old, b).start()
        extract(t_new, b)
        gcp(b).start()
      return carry

    lax.fori_loop(1, _T // _NB, main_body, 0)

    for b in range(_NB):
      t_old = _T - _NB + b
      gcp(b).wait()
      scp(t_old, b).wait()
      transpose(b)
      scp(t_old, b).start()
    for b in range(_NB):
      scp(_T - _NB + b, b).wait()

  return emb


@jax.jit
def kernel(x, weight):
  flat = x.reshape(-1)
  out5 = _build()(flat, weight)
  return out5.transpose(2, 4, 0, 1, 3).reshape(_B, _T, _D)


# speedup vs baseline: 2.9609x; 1.4437x over previous
import functools

import jax
import jax.numpy as jnp
from jax import lax
from jax.experimental import pallas as pl
from jax.experimental.pallas import tpu as pltpu
from jax.experimental.pallas import tpu_sc as plsc

_B, _T = 4096, 200
_V, _D = 1000000, 32
_NW = 32
_BB = 128
_NB = 4


def _build_repack():
  mesh = plsc.VectorSubcoreMesh(core_axis_name="c", subcore_axis_name="s")
  n_full = 7812

  @functools.partial(
      pl.kernel,
      mesh=mesh,
      out_type=jax.ShapeDtypeStruct((_V // 4, 128), jnp.float32),
      scratch_types=[
          pltpu.VMEM((2, _D, 128), jnp.float32),
          pltpu.VMEM((2, _D, 128), jnp.float32),
          pltpu.SemaphoreType.DMA((2,)),
          pltpu.SemaphoreType.DMA((2,)),
      ],
      compiler_params=pltpu.CompilerParams(
          use_tc_tiling_on_sc=True, needs_layout_passes=False),
  )
  def rep(wt_hbm, tail_hbm, out_hbm, in_v, ob_v, isem, osem):
    wid = lax.axis_index("s") * 2 + lax.axis_index("c")
    n_w = 244 + jnp.where(wid < 4, 1, 0)
    base = wid * 244 + jnp.minimum(wid, 4)
    lanes = lax.iota(jnp.int32, 16)

    def icp(blk, s):
      off = pl.multiple_of(blk * 128, 128)
      return pltpu.make_async_copy(
          wt_hbm.at[:, pl.ds(off, 128)], in_v.at[s], isem.at[s])

    def ocp(blk, s):
      off = pl.multiple_of(blk * 32, 32)
      return pltpu.make_async_copy(
          ob_v.at[s], out_hbm.at[pl.ds(off, 32), :], osem.at[s])

    def transpose(s, njg):
      def body(d0, carry):
        dvec = jnp.bitwise_and(d0 + lanes, _D - 1)
        for k in range(njg):
          jvec = k * 16 + lanes
          vals = plsc.load_gather(in_v.at[s], [dvec, jvec])
          f = jvec * _D + dvec
          plsc.store_scatter(
              ob_v.at[s],
              [lax.shift_right_logical(f, 7), jnp.bitwise_and(f, 127)],
              vals)
        return carry
      lax.fori_loop(0, _D, body, 0)

    icp(base, 0).start()
    icp(base + 1, 1).start()

    def pair_body(p, carry):
      for s in range(2):
        i = p * 2 + s
        blk = base + i
        icp(blk, s).wait()

        @pl.when(i + 2 < n_w)
        def _():
          icp(blk + 2, s).start()

        @pl.when(i >= 2)
        def _():
          ocp(blk - 2, s).wait()

        transpose(s, 8)
        ocp(blk, s).start()
      return carry

    lax.fori_loop(0, 122, pair_body, 0)

    @pl.when(wid < 4)
    def _():
      icp(base + 244, 0).wait()
      ocp(base + 242, 0).wait()
      transpose(0, 8)
      ocp(base + 244, 0).start()

    @pl.when(wid < 4)
    def _():
      ocp(base + 244, 0).wait()

    @pl.when(wid >= 4)
    def _():
      ocp(base + 242, 0).wait()
    ocp(base + 243, 1).wait()

    @pl.when(wid == 31)
    def _():
      pltpu.sync_copy(tail_hbm, in_v.at[0])
      transpose(0, 4)
      pltpu.sync_copy(
          ob_v.at[0, pl.ds(0, 16), :],
          out_hbm.at[pl.ds(n_full * 32, 16), :])

  return rep


def _build():
  mesh = plsc.VectorSubcoreMesh(core_axis_name="c", subcore_axis_name="s")
  per_w = _BB * _T

  @functools.partial(
      pl.kernel,
      mesh=mesh,
      out_type=jax.ShapeDtypeStruct((_T, 4, _NW, 8, 128), jnp.float32),
      scratch_types=[
          pltpu.VMEM((per_w,), jnp.int32),
          pltpu.VMEM((_NB, _BB, _D), jnp.float32),
          pltpu.VMEM((_NB, 4, 8, 128), jnp.float32),
          pltpu.VMEM((_NB, _BB), jnp.int32),
          pltpu.SemaphoreType.DMA((_NB,)),
          pltpu.SemaphoreType.DMA((_NB,)),
      ],
      compiler_params=pltpu.CompilerParams(
          use_tc_tiling_on_sc=False, needs_layout_passes=False),
  )
  def emb(idx_hbm, w_hbm, out_hbm, idx_v, g_v, tr_v, il_v, gsem, ssem):
    wid = lax.axis_index("s") * 2 + lax.axis_index("c")
    pltpu.sync_copy(idx_hbm.at[pl.ds(wid * per_w, per_w)], idx_v)
    lanes = lax.iota(jnp.int32, 16)

    def gcp(b):
      return pltpu.make_async_copy(
          w_hbm.at[il_v.at[b]], g_v.at[b], gsem.at[b])

    def scp(t, b):
      return pltpu.make_async_copy(
          tr_v.at[b], out_hbm.at[t, :, wid], ssem.at[b])

    def extract(t, b):
      for k in range(8):
        pos = (k * 16 + lanes) * _T + t
        il_v[b, pl.ds(k * 16, 16)] = plsc.load_gather(idx_v, [pos])

    def transpose(b):
      def body(d0, carry):
        dvec = jnp.bitwise_and(d0 + lanes, _D - 1)
        i0 = lax.shift_right_logical(dvec, 3)
        i1 = jnp.bitwise_and(dvec, 7)
        for k in range(8):
          jvec = k * 16 + lanes
          vals = plsc.load_gather(g_v.at[b], [jvec, dvec])
          plsc.store_scatter(tr_v.at[b], [i0, i1, jvec], vals)
        return carry
      lax.fori_loop(0, _D, body, 0)

    for b in range(_NB):
      extract(b, b)
      gcp(b).start()

    def main_body(r, carry):
      for b in range(_NB):
        t_new = r * _NB + b
        t_old = t_new - _NB
        gcp(b).wait()

        @pl.when(r >= 2)
        def _():
          scp(t_old, b).wait()

        transpose(b)
        scp(t_old, b).start()
        extract(t_new, b)
        gcp(b).start()
      return carry

    lax.fori_loop(1, _T // _NB, main_body, 0)

    for b in range(_NB):
      t_old = _T - _NB + b
      gcp(b).wait()
      scp(t_old, b).wait()
      transpose(b)
      scp(t_old, b).start()
    for b in range(_NB):
      scp(_T - _NB + b, b).wait()

  return emb


@jax.jit
def kernel(x, weight):
  flat = x.reshape(-1)
  wt = jnp.swapaxes(weight, 0, 1)
  tail = jnp.pad(wt[:, 7812 * 128:], ((0, 0), (0, 64)))
  packed = _build_repack()(wt, tail)
  out5 = _build()(flat, packed.reshape(_V, _D))
  return out5.transpose(2, 4, 0, 1, 3).reshape(_B, _T, _D)
